# Initial kernel scaffold; baseline (speedup 1.0000x reference)
import jax
import jax.numpy as jnp
from jax.experimental import pallas as pl


def kernel(x, token_table, pos_table):
    raise NotImplementedError("write your pallas kernel here")


# speedup vs baseline: 4.2610x; 4.2610x over previous
import functools

import jax
import jax.numpy as jnp
from jax import lax
from jax.experimental import pallas as pl
from jax.experimental.pallas import tpu as pltpu
from jax.experimental.pallas import tpu_sc as plsc

VOCAB = 100000
L = 200
D = 128
B = 4096

NC, NS = 2, 16
NW = NC * NS
SEQ_PER_W = B // NW
GCHUNK = 40
NCHUNK = L // GCHUNK


def _body(tok_hbm, x_hbm, pos_hbm, out_hbm, idx_v, pos_v, rows_v, sem):
    wid = lax.axis_index("s") * NC + lax.axis_index("c")
    pltpu.sync_copy(pos_hbm, pos_v)

    def seq_body(s, carry):
        seqbase = (wid * SEQ_PER_W + s) * L
        pltpu.sync_copy(x_hbm.at[pl.ds(seqbase, L)], idx_v)
        copies = [
            pltpu.async_copy(
                tok_hbm.at[idx_v.at[pl.ds(c * GCHUNK, GCHUNK)]],
                rows_v.at[pl.ds(c * GCHUNK, GCHUNK)],
                sem,
            )
            for c in range(NCHUNK)
        ]
        for cp in copies:
            cp.wait()

        def add_body(r, c2):
            for j in range(D // 16):
                sl = pl.ds(j * 16, 16)
                plsc.addupdate(rows_v.at[r, sl], pos_v[r, sl])
            return c2

        lax.fori_loop(0, L, add_body, 0)
        base = (wid * SEQ_PER_W + s) * L
        pltpu.sync_copy(rows_v, out_hbm.at[pl.ds(base, L)])
        return carry

    lax.fori_loop(0, SEQ_PER_W, seq_body, 0)


def kernel(x, token_table, pos_table):
    x = x.astype(jnp.int32)
    mesh = plsc.VectorSubcoreMesh(core_axis_name="c", subcore_axis_name="s")
    run = functools.partial(
        pl.kernel,
        mesh=mesh,
        out_type=jax.ShapeDtypeStruct((B * L, D), jnp.float32),
        scratch_types=[
            pltpu.VMEM((L,), jnp.int32),
            pltpu.VMEM((L, D), jnp.float32),
            pltpu.VMEM((L, D), jnp.float32),
            pltpu.SemaphoreType.DMA,
        ],
    )(_body)
    out = run(token_table, x.reshape(B * L), pos_table)
    return out.reshape(B, L, D)


# speedup vs baseline: 7.5224x; 1.7654x over previous
import functools

import jax
import jax.numpy as jnp
from jax import lax
from jax.experimental import pallas as pl
from jax.experimental.pallas import tpu as pltpu
from jax.experimental.pallas import tpu_sc as plsc

VOCAB = 100000
L = 200
D = 128
B = 4096

NC, NS = 2, 16
NW = NC * NS
SEQ_PER_W = B // NW
CHUNKS = ((0, 128), (128, 72))


def _body(tok_hbm, x_hbm, pos_hbm, out_hbm,
          idx_v, pos_v, rows0, rows1, gsem, ssem0, ssem1):
    wid = lax.axis_index("s") * NC + lax.axis_index("c")
    rows = (rows0, rows1)
    ssem = (ssem0, ssem1)
    pltpu.sync_copy(pos_hbm, pos_v)
    nidx = SEQ_PER_W * L
    pltpu.sync_copy(x_hbm.at[pl.ds(wid * nidx, nidx)], idx_v)

    def gather_descs(s, buf):
        return [
            pltpu.make_async_copy(
                tok_hbm.at[idx_v.at[pl.ds(s * L + off, ln)]],
                buf.at[pl.ds(off, ln)],
                gsem,
            )
            for off, ln in CHUNKS
        ]

    for cp in gather_descs(0, rows0):
        cp.start()
    for cp in gather_descs(0, rows0):
        cp.wait()

    def pair_body(g, carry):
        for b in range(2):
            s = 2 * g + b
            nb = 1 - b

            @pl.when(s >= 1)
            def _():
                pltpu.make_async_copy(
                    rows[nb], out_hbm.at[pl.ds(0, L)], ssem[nb]).wait()

            @pl.when(s + 1 < SEQ_PER_W)
            def _():
                for cp in gather_descs(s + 1, rows[nb]):
                    cp.start()

            def add_body(r, c2):
                for j in range(D // 16):
                    sl = pl.ds(j * 16, 16)
                    plsc.addupdate(rows[b].at[r, sl], pos_v[r, sl])
                return c2

            lax.fori_loop(0, L, add_body, 0)

            base = (wid * SEQ_PER_W + s) * L
            pltpu.async_copy(rows[b], out_hbm.at[pl.ds(base, L)], ssem[b])

            @pl.when(s + 1 < SEQ_PER_W)
            def _():
                for cp in gather_descs(s + 1, rows[nb]):
                    cp.wait()
        return carry

    lax.fori_loop(0, SEQ_PER_W // 2, pair_body, 0)
    pltpu.make_async_copy(rows[1], out_hbm.at[pl.ds(0, L)], ssem[1]).wait()


def kernel(x, token_table, pos_table):
    x = x.astype(jnp.int32)
    mesh = plsc.VectorSubcoreMesh(core_axis_name="c", subcore_axis_name="s")
    run = functools.partial(
        pl.kernel,
        mesh=mesh,
        out_type=jax.ShapeDtypeStruct((B * L, D), jnp.float32),
        scratch_types=[
            pltpu.VMEM((SEQ_PER_W * L,), jnp.int32),
            pltpu.VMEM((L, D), jnp.float32),
            pltpu.VMEM((L, D), jnp.float32),
            pltpu.VMEM((L, D), jnp.float32),
            pltpu.SemaphoreType.DMA,
            pltpu.SemaphoreType.DMA,
            pltpu.SemaphoreType.DMA,
        ],
    )(_body)
    out = run(token_table, x.reshape(B * L), pos_table)
    return out.reshape(B, L, D)
